# Initial kernel scaffold; baseline (speedup 1.0000x reference)
import jax
import jax.numpy as jnp
from jax.experimental import pallas as pl


def kernel(logits, sem_gt, is_train, previous_features, previous_count, var):
    raise NotImplementedError("write your pallas kernel here")


# speedup vs baseline: 46.9986x; 46.9986x over previous
import jax
import jax.numpy as jnp
from jax.experimental import pallas as pl
from jax.experimental.pallas import tpu as pltpu

_N = 19
_DELTA = 0.1
_PX = 32768


def _body(pf_ref, pc_ref, var_ref, lg_ref, lab_ref, out_ref, sums_ref, cnts_ref):
    b = pl.program_id(0)
    j = pl.program_id(1)
    nb = pl.num_programs(0)
    nj = pl.num_programs(1)

    @pl.when((b == 0) & (j == 0))
    def _init():
        sums_ref[...] = jnp.zeros_like(sums_ref)
        cnts_ref[...] = jnp.zeros_like(cnts_ref)

    var = var_ref[...]
    pos = var > 0
    nzmin = jnp.min(jnp.where(pos, jnp.abs(var), jnp.inf), axis=1, keepdims=True)
    variance = jnp.where(pos, nzmin, var)
    inv_nv = 1.0 / (variance / nzmin + 1e-8)

    lab = lab_ref[0]
    ohT = (jax.lax.broadcasted_iota(jnp.int32, (_N, _PX), 0) == lab).astype(
        jnp.float32
    )
    dn = (((0,), (0,)), ((), ()))
    mavT = jax.lax.dot_general(pf_ref[...], ohT, dn, preferred_element_type=jnp.float32)
    ivT = jax.lax.dot_general(inv_nv, ohT, dn, preferred_element_type=jnp.float32)
    ew = jnp.abs(lg_ref[0] - mavT) * ivT
    ew = jnp.maximum(ew - _DELTA, 0.0)
    per_px = jnp.sum(ew, axis=0, keepdims=True)
    dn_px = (((1,), (1,)), ((), ()))
    s = jax.lax.dot_general(ohT, per_px, dn_px, preferred_element_type=jnp.float32)
    c = jnp.sum(ohT, axis=1, keepdims=True)
    sums_ref[...] += jnp.broadcast_to(s, sums_ref.shape)
    cnts_ref[...] += jnp.broadcast_to(c, cnts_ref.shape)

    @pl.when((b == nb - 1) & (j == nj - 1))
    def _fin():
        sums = sums_ref[...]
        cnts = cnts_ref[...]
        means = sums / jnp.maximum(cnts * float(_N), 1.0)
        varsum = jnp.sum(var, axis=1, keepdims=True)
        krow = jax.lax.broadcasted_iota(jnp.int32, (_N, 128), 0)
        valid = (pc_ref[...] > 0.0) & (varsum != 0.0) & (cnts > 0.0) & (krow > 0)
        lane0 = jax.lax.broadcasted_iota(jnp.int32, (_N, 128), 1) == 0
        out_ref[0, 0] = jnp.sum(jnp.where(valid & lane0, means, 0.0))


def kernel(logits, sem_gt, is_train, previous_features, previous_count, var):
    del is_train
    B, C, H, W = logits.shape
    hw = H * W
    lg = logits.reshape(B, C, hw)
    lab = sem_gt.reshape(B, 1, hw)
    pc = previous_count.reshape(_N, 1)
    grid = (B, hw // _PX)
    out = pl.pallas_call(
        _body,
        grid=grid,
        in_specs=[
            pl.BlockSpec((_N, _N), lambda b, j: (0, 0)),
            pl.BlockSpec((_N, 1), lambda b, j: (0, 0)),
            pl.BlockSpec((_N, _N), lambda b, j: (0, 0)),
            pl.BlockSpec((1, _N, _PX), lambda b, j: (b, 0, j)),
            pl.BlockSpec((1, 1, _PX), lambda b, j: (b, 0, j)),
        ],
        out_specs=pl.BlockSpec(memory_space=pltpu.SMEM),
        out_shape=jax.ShapeDtypeStruct((1, 1), jnp.float32),
        scratch_shapes=[
            pltpu.VMEM((_N, 128), jnp.float32),
            pltpu.VMEM((_N, 128), jnp.float32),
        ],
    )(previous_features, pc, var, lg, lab)
    return out[0, 0]
